# baseline (speedup 1.0000x reference)
import functools

import jax
import jax.numpy as jnp
from jax import lax
from jax.experimental import pallas as pl
from jax.experimental.pallas import tpu as pltpu
from jax.experimental.pallas import tpu_sc as plsc

E = 64
K = 8
_T = 256
_CH = 4

_NC = 2
_NS = 16
_NTILE = _NC * _NS
_L = 16

_NEG = float("-inf")


def _logits_body(x_ref, w_ref, b_ref, out_ref):
    lg = lax.dot_general(
        w_ref[...], x_ref[...], (((1,), (1,)), ((), ())),
        preferred_element_type=jnp.float32)
    lg = lg + b_ref[...]
    tpb = out_ref.shape[2]
    for s in range(out_ref.shape[0]):
        out_ref[s] = lg[:, s * tpb:(s + 1) * tpb]


def _tc_logits(xc, W, b2, tpb):
    nch, D = xc.shape
    grid = nch // _T
    tiles_per_step = _T // tpb
    return pl.pallas_call(
        _logits_body,
        grid=(grid,),
        in_specs=[
            pl.BlockSpec((_T, D), lambda i: (i, 0)),
            pl.BlockSpec((E, D), lambda i: (0, 0)),
            pl.BlockSpec((E, 1), lambda i: (0, 0)),
        ],
        out_specs=pl.BlockSpec(
            (tiles_per_step, E, tpb), lambda i: (i, 0, 0)),
        out_shape=jax.ShapeDtypeStruct((_NTILE, E, tpb), jnp.float32),
        compiler_params=pltpu.CompilerParams(
            dimension_semantics=("parallel",),
        ),
    )(xc, W, b2)


def _sc_router_body(tpb, lg_hbm, w_hbm, mask_hbm, idx_hbm,
                    slab, maskv, wv, iv, sem):
    wid = lax.axis_index("s") * _NC + lax.axis_index("c")
    cp = pltpu.make_async_copy(lg_hbm.at[wid], slab, sem)
    cp.start()
    zero = jnp.zeros((_L,), jnp.float32)

    def zbody(r, c):
        for j in range(8):
            maskv[pl.ds(r * 8 * _L + j * _L, _L)] = zero
        return c
    lax.fori_loop(0, tpb * E // (8 * _L), zbody, 0)
    cp.wait()

    toki = lax.broadcasted_iota(jnp.int32, (_L,), 0)
    negv = jnp.full((_L,), _NEG, jnp.float32)
    zeroi = jnp.zeros((_L,), jnp.int32)

    def group(g, c):
        goff = g * _L
        tokrel = goff + toki
        ms, mis = [], []
        for k in range(K):
            def scan(eb, carry):
                m, mi = carry
                for j in range(8):
                    e = eb * 8 + j
                    v = slab[pl.ds(e * tpb + goff, _L)]
                    gt = v > m
                    m = jnp.where(gt, v, m)
                    mi = jnp.where(gt, jnp.full((_L,), e, jnp.int32), mi)
                return (m, mi)
            m, mi = lax.fori_loop(0, E // 8, scan, (negv, zeroi))
            ms.append(m)
            mis.append(mi)
            if k < K - 1:
                plsc.store_scatter(slab, [mi * tpb + tokrel], negv)
        es = [jnp.exp(m - ms[0]) for m in ms]
        s = es[0]
        for k in range(1, K):
            s = s + es[k]
        r = 1.0 / s
        tk = tokrel * K
        for k in range(K):
            wk = es[k] * r
            plsc.store_scatter(wv, [tk + k], wk)
            plsc.store_scatter(iv, [tk + k], mis[k])
            plsc.store_scatter(maskv, [tokrel * E + mis[k]], wk)
        return c
    lax.fori_loop(0, tpb // _L, group, 0)

    pltpu.sync_copy(wv, w_hbm.at[wid])
    pltpu.sync_copy(maskv, mask_hbm.at[wid])
    pltpu.sync_copy(iv, idx_hbm.at[wid])


def _sc_router(lg3, tpb):
    mesh = plsc.VectorSubcoreMesh(core_axis_name="c", subcore_axis_name="s")
    return pl.kernel(
        functools.partial(_sc_router_body, tpb),
        out_type=[
            jax.ShapeDtypeStruct((_NTILE, tpb * K), jnp.float32),
            jax.ShapeDtypeStruct((_NTILE, tpb * E), jnp.float32),
            jax.ShapeDtypeStruct((_NTILE, tpb * K), jnp.int32),
        ],
        mesh=mesh,
        compiler_params=pltpu.CompilerParams(needs_layout_passes=False),
        scratch_types=[
            pltpu.VMEM((E * tpb,), jnp.float32),
            pltpu.VMEM((tpb * E,), jnp.float32),
            pltpu.VMEM((tpb * K,), jnp.float32),
            pltpu.VMEM((tpb * K,), jnp.int32),
            pltpu.SemaphoreType.DMA,
        ],
    )(lg3.reshape(_NTILE, E * tpb))


@jax.jit
def kernel(x, W, b):
    B, S, D = x.shape
    n = B * S
    nch = n // _CH
    tpb = nch // _NTILE
    xf = x.reshape(n, D)
    b2 = b.reshape(E, 1)
    ws, masks, idxs = [], [], []
    for c in range(_CH):
        lg3 = _tc_logits(lax.slice(xf, (c * nch, 0), ((c + 1) * nch, D)),
                         W, b2, tpb)
        w3, mask3, idx3 = _sc_router(lg3, tpb)
        ws.append(w3)
        masks.append(mask3)
        idxs.append(idx3)
    w = jnp.concatenate(ws, axis=0).reshape(B, S, K)
    mask = jnp.concatenate(masks, axis=0).reshape(B, S, E)
    idx = jnp.concatenate(idxs, axis=0).reshape(B, S, K)
    return (w, mask, idx)


# speedup vs baseline: 1.8785x; 1.8785x over previous
import functools

import jax
import jax.numpy as jnp
from jax import lax
from jax.experimental import pallas as pl
from jax.experimental.pallas import tpu as pltpu
from jax.experimental.pallas import tpu_sc as plsc

E = 64
K = 8
_T = 256
_CH = 4

_NC = 2
_NS = 16
_NTILE = _NC * _NS
_L = 16

_NEG = float("-inf")


def _logits_body(x_ref, w_ref, b_ref, out_ref):
    lg = lax.dot_general(
        w_ref[...], x_ref[...], (((1,), (1,)), ((), ())),
        preferred_element_type=jnp.float32)
    lg = lg + b_ref[...]
    tpb = out_ref.shape[2]
    for s in range(out_ref.shape[0]):
        out_ref[s] = lg[:, s * tpb:(s + 1) * tpb]


def _tc_logits(xf, W, b2, tpb, chunk, nch):
    n, D = xf.shape
    grid = nch // _T
    step0 = chunk * grid
    tiles_per_step = _T // tpb
    return pl.pallas_call(
        _logits_body,
        grid=(grid,),
        in_specs=[
            pl.BlockSpec((_T, D), lambda i: (step0 + i, 0)),
            pl.BlockSpec((E, D), lambda i: (0, 0)),
            pl.BlockSpec((E, 1), lambda i: (0, 0)),
        ],
        out_specs=pl.BlockSpec(
            (tiles_per_step, E, tpb), lambda i: (i, 0, 0)),
        out_shape=jax.ShapeDtypeStruct((_NTILE, E, tpb), jnp.float32),
        compiler_params=pltpu.CompilerParams(
            dimension_semantics=("parallel",),
        ),
    )(xf, W, b2)


def _sc_router_body(tpb, lg_hbm, w_hbm, mask_hbm, idx_hbm,
                    slab, maskv, wv, iv, sem):
    wid = lax.axis_index("s") * _NC + lax.axis_index("c")
    cp = pltpu.make_async_copy(lg_hbm.at[wid], slab, sem)
    cp.start()
    zero = jnp.zeros((_L,), jnp.float32)

    def zbody(r, c):
        for j in range(8):
            maskv[pl.ds(r * 8 * _L + j * _L, _L)] = zero
        return c
    lax.fori_loop(0, tpb * E // (8 * _L), zbody, 0)
    cp.wait()

    toki = lax.broadcasted_iota(jnp.int32, (_L,), 0)
    negv = jnp.full((_L,), _NEG, jnp.float32)
    zeroi = jnp.zeros((_L,), jnp.int32)

    def group(g, c):
        goff = g * _L
        tokrel = goff + toki
        ms, mis = [], []
        for k in range(K):
            def scan(eb, carry):
                m, mi = carry
                for j in range(8):
                    e = eb * 8 + j
                    v = slab[pl.ds(e * tpb + goff, _L)]
                    gt = v > m
                    m = jnp.where(gt, v, m)
                    mi = jnp.where(gt, jnp.full((_L,), e, jnp.int32), mi)
                return (m, mi)
            m, mi = lax.fori_loop(0, E // 8, scan, (negv, zeroi))
            ms.append(m)
            mis.append(mi)
            if k < K - 1:
                plsc.store_scatter(slab, [mi * tpb + tokrel], negv)
        es = [jnp.exp(m - ms[0]) for m in ms]
        s = es[0]
        for k in range(1, K):
            s = s + es[k]
        r = 1.0 / s
        tk = tokrel * K
        for k in range(K):
            wk = es[k] * r
            plsc.store_scatter(wv, [tk + k], wk)
            plsc.store_scatter(iv, [tk + k], mis[k])
            plsc.store_scatter(maskv, [tokrel * E + mis[k]], wk)
        return c
    lax.fori_loop(0, tpb // _L, group, 0)

    pltpu.sync_copy(wv, w_hbm.at[wid])
    pltpu.sync_copy(maskv, mask_hbm.at[wid])
    pltpu.sync_copy(iv, idx_hbm.at[wid])


def _sc_router(lg3, tpb):
    mesh = plsc.VectorSubcoreMesh(core_axis_name="c", subcore_axis_name="s")
    return pl.kernel(
        functools.partial(_sc_router_body, tpb),
        out_type=[
            jax.ShapeDtypeStruct((_NTILE, tpb * K), jnp.float32),
            jax.ShapeDtypeStruct((_NTILE, tpb * E), jnp.float32),
            jax.ShapeDtypeStruct((_NTILE, tpb * K), jnp.int32),
        ],
        mesh=mesh,
        compiler_params=pltpu.CompilerParams(needs_layout_passes=False),
        scratch_types=[
            pltpu.VMEM((E * tpb,), jnp.float32),
            pltpu.VMEM((tpb * E,), jnp.float32),
            pltpu.VMEM((tpb * K,), jnp.float32),
            pltpu.VMEM((tpb * K,), jnp.int32),
            pltpu.SemaphoreType.DMA,
        ],
    )(lg3.reshape(_NTILE, E * tpb))


@jax.jit
def kernel(x, W, b):
    B, S, D = x.shape
    n = B * S
    nch = n // _CH
    tpb = nch // _NTILE
    xf = x.reshape(n, D)
    b2 = b.reshape(E, 1)
    ws, masks, idxs = [], [], []
    for c in range(_CH):
        lg3 = _tc_logits(xf, W, b2, tpb, c, nch)
        w3, mask3, idx3 = _sc_router(lg3, tpb)
        ws.append(w3)
        masks.append(mask3)
        idxs.append(idx3)
    w = jnp.concatenate(ws, axis=0).reshape(B, S, K)
    mask = jnp.concatenate(masks, axis=0).reshape(B, S, E)
    idx = jnp.concatenate(idxs, axis=0).reshape(B, S, K)
    return (w, mask, idx)


# speedup vs baseline: 1.8870x; 1.0045x over previous
import functools

import jax
import jax.numpy as jnp
from jax import lax
from jax.experimental import pallas as pl
from jax.experimental.pallas import tpu as pltpu
from jax.experimental.pallas import tpu_sc as plsc

E = 64
K = 8
_T = 256
_CH = 4

_NC = 2
_NS = 16
_NTILE = _NC * _NS
_L = 16
_NSEG = 8
_SEGW = E // _NSEG

_NEG = float("-inf")


def _logits_body(x_ref, w_ref, b_ref, out_ref):
    lg = lax.dot_general(
        w_ref[...], x_ref[...], (((1,), (1,)), ((), ())),
        preferred_element_type=jnp.float32)
    lg = lg + b_ref[...]
    tpb = out_ref.shape[2]
    for s in range(out_ref.shape[0]):
        out_ref[s] = lg[:, s * tpb:(s + 1) * tpb]


def _tc_logits(xf, W, b2, tpb, chunk, nch):
    n, D = xf.shape
    grid = nch // _T
    step0 = chunk * grid
    tiles_per_step = _T // tpb
    return pl.pallas_call(
        _logits_body,
        grid=(grid,),
        in_specs=[
            pl.BlockSpec((_T, D), lambda i: (step0 + i, 0)),
            pl.BlockSpec((E, D), lambda i: (0, 0)),
            pl.BlockSpec((E, 1), lambda i: (0, 0)),
        ],
        out_specs=pl.BlockSpec(
            (tiles_per_step, E, tpb), lambda i: (i, 0, 0)),
        out_shape=jax.ShapeDtypeStruct((_NTILE, E, tpb), jnp.float32),
        compiler_params=pltpu.CompilerParams(
            dimension_semantics=("parallel",),
        ),
    )(xf, W, b2)


def _sc_router_body(tpb, lg_hbm, w_hbm, mask_hbm, idx_hbm,
                    slab, maskv, wv, iv, sem):
    wid = lax.axis_index("s") * _NC + lax.axis_index("c")
    cp = pltpu.make_async_copy(lg_hbm.at[wid], slab, sem)
    cp.start()
    zero = jnp.zeros((_L,), jnp.float32)

    def zbody(r, c):
        for j in range(8):
            maskv[pl.ds(r * 8 * _L + j * _L, _L)] = zero
        return c
    lax.fori_loop(0, tpb * E // (8 * _L), zbody, 0)
    cp.wait()

    toki = lax.broadcasted_iota(jnp.int32, (_L,), 0)
    negv = jnp.full((_L,), _NEG, jnp.float32)
    zeroi = jnp.zeros((_L,), jnp.int32)

    def group(g, c):
        goff = g * _L
        tokrel = goff + toki
        seg_m, seg_mi = [], []
        for s in range(_NSEG):
            m, mi = negv, zeroi
            for j in range(_SEGW):
                e = s * _SEGW + j
                v = slab[pl.ds(e * tpb + goff, _L)]
                gt = v > m
                m = jnp.where(gt, v, m)
                mi = jnp.where(gt, jnp.full((_L,), e, jnp.int32), mi)
            seg_m.append(m)
            seg_mi.append(mi)

        ms, mis = [], []
        for k in range(K):
            m, mi, sseg = seg_m[0], seg_mi[0], zeroi
            for s in range(1, _NSEG):
                gt = seg_m[s] > m
                m = jnp.where(gt, seg_m[s], m)
                mi = jnp.where(gt, seg_mi[s], mi)
                sseg = jnp.where(gt, jnp.full((_L,), s, jnp.int32), sseg)
            ms.append(m)
            mis.append(mi)
            if k < K - 1:
                plsc.store_scatter(slab, [mi * tpb + tokrel], negv)
                nm, nmi = negv, zeroi
                sbase = sseg * _SEGW
                gbase = sbase * tpb + tokrel
                for j in range(_SEGW):
                    v = plsc.load_gather(slab, [gbase + j * tpb])
                    gt = v > nm
                    nm = jnp.where(gt, v, nm)
                    nmi = jnp.where(gt, sbase + j, nmi)
                for s in range(_NSEG):
                    hit = sseg == s
                    seg_m[s] = jnp.where(hit, nm, seg_m[s])
                    seg_mi[s] = jnp.where(hit, nmi, seg_mi[s])
        es = [jnp.exp(m - ms[0]) for m in ms]
        ssum = es[0]
        for k in range(1, K):
            ssum = ssum + es[k]
        r = 1.0 / ssum
        tk = tokrel * K
        for k in range(K):
            wk = es[k] * r
            plsc.store_scatter(wv, [tk + k], wk)
            plsc.store_scatter(iv, [tk + k], mis[k])
            plsc.store_scatter(maskv, [tokrel * E + mis[k]], wk)
        return c
    lax.fori_loop(0, tpb // _L, group, 0)

    pltpu.sync_copy(wv, w_hbm.at[pl.ds(wid * tpb * K, tpb * K)])
    pltpu.sync_copy(maskv, mask_hbm.at[pl.ds(wid * tpb * E, tpb * E)])
    pltpu.sync_copy(iv, idx_hbm.at[pl.ds(wid * tpb * K, tpb * K)])


def _sc_router(lg3, tpb):
    mesh = plsc.VectorSubcoreMesh(core_axis_name="c", subcore_axis_name="s")
    return pl.kernel(
        functools.partial(_sc_router_body, tpb),
        out_type=[
            jax.ShapeDtypeStruct((_NTILE * tpb * K,), jnp.float32),
            jax.ShapeDtypeStruct((_NTILE * tpb * E,), jnp.float32),
            jax.ShapeDtypeStruct((_NTILE * tpb * K,), jnp.int32),
        ],
        mesh=mesh,
        compiler_params=pltpu.CompilerParams(needs_layout_passes=False),
        scratch_types=[
            pltpu.VMEM((E * tpb,), jnp.float32),
            pltpu.VMEM((tpb * E,), jnp.float32),
            pltpu.VMEM((tpb * K,), jnp.float32),
            pltpu.VMEM((tpb * K,), jnp.int32),
            pltpu.SemaphoreType.DMA,
        ],
    )(lg3.reshape(_NTILE, E * tpb))


@jax.jit
def kernel(x, W, b):
    B, S, D = x.shape
    n = B * S
    nch = n // _CH
    tpb = nch // _NTILE
    xf = x.reshape(n, D)
    b2 = b.reshape(E, 1)
    ws, masks, idxs = [], [], []
    for c in range(_CH):
        lg3 = _tc_logits(xf, W, b2, tpb, c, nch)
        w3, mask3, idx3 = _sc_router(lg3, tpb)
        ws.append(w3)
        masks.append(mask3)
        idxs.append(idx3)
    w = jnp.concatenate(ws).reshape(B, S, K)
    mask = jnp.concatenate(masks).reshape(B, S, E)
    idx = jnp.concatenate(idxs).reshape(B, S, K)
    return (w, mask, idx)


# speedup vs baseline: 2.4622x; 1.3049x over previous
import functools

import jax
import jax.numpy as jnp
from jax import lax
from jax.experimental import pallas as pl
from jax.experimental.pallas import tpu as pltpu
from jax.experimental.pallas import tpu_sc as plsc

E = 64
K = 8
_T = 256
_CH = 4

_NC = 2
_NS = 16
_NTILE = _NC * _NS
_L = 16
_NSEG = 8
_SEGW = E // _NSEG

_NEG = float("-inf")


def _logits_body(x_ref, w_ref, b_ref, out_ref):
    lg = lax.dot_general(
        w_ref[...], x_ref[...], (((1,), (1,)), ((), ())),
        preferred_element_type=jnp.float32)
    lg = lg + b_ref[...]
    tpb = out_ref.shape[2]
    for s in range(out_ref.shape[0]):
        out_ref[s] = lg[:, s * tpb:(s + 1) * tpb]


def _tc_logits(xf, W, b2, tpb, chunk, nch):
    n, D = xf.shape
    grid = nch // _T
    step0 = chunk * grid
    tiles_per_step = _T // tpb
    return pl.pallas_call(
        _logits_body,
        grid=(grid,),
        in_specs=[
            pl.BlockSpec((_T, D), lambda i: (step0 + i, 0)),
            pl.BlockSpec((E, D), lambda i: (0, 0)),
            pl.BlockSpec((E, 1), lambda i: (0, 0)),
        ],
        out_specs=pl.BlockSpec(
            (tiles_per_step, E, tpb), lambda i: (i, 0, 0)),
        out_shape=jax.ShapeDtypeStruct((_NTILE, E, tpb), jnp.float32),
        compiler_params=pltpu.CompilerParams(
            dimension_semantics=("parallel",),
        ),
    )(xf, W, b2)


def _sc_router_body(tpb, lg_hbm, w_hbm, mask_hbm, idx_hbm,
                    slab, maskv, wv, iv, sem):
    wid = lax.axis_index("s") * _NC + lax.axis_index("c")
    cp = pltpu.make_async_copy(lg_hbm.at[wid], slab, sem)
    cp.start()
    zero = jnp.zeros((_L,), jnp.float32)

    def zbody(r, c):
        for j in range(8):
            maskv[pl.ds(r * 8 * _L + j * _L, _L)] = zero
        return c
    lax.fori_loop(0, tpb * E // (8 * _L), zbody, 0)
    cp.wait()

    toki = lax.broadcasted_iota(jnp.int32, (_L,), 0)
    negv = jnp.full((_L,), _NEG, jnp.float32)
    zeroi = jnp.zeros((_L,), jnp.int32)

    def group(g, c):
        goff = g * _L
        tokrel = goff + toki
        seg_m, seg_mi = [], []
        for s in range(_NSEG):
            m, mi = negv, zeroi
            for j in range(_SEGW):
                e = s * _SEGW + j
                v = slab[e, pl.ds(goff, _L)]
                gt = v > m
                m = jnp.where(gt, v, m)
                mi = jnp.where(gt, jnp.full((_L,), e, jnp.int32), mi)
            seg_m.append(m)
            seg_mi.append(mi)

        ms, mis = [], []
        for k in range(K):
            m, mi, sseg = seg_m[0], seg_mi[0], zeroi
            for s in range(1, _NSEG):
                gt = seg_m[s] > m
                m = jnp.where(gt, seg_m[s], m)
                mi = jnp.where(gt, seg_mi[s], mi)
                sseg = jnp.where(gt, jnp.full((_L,), s, jnp.int32), sseg)
            ms.append(m)
            mis.append(mi)
            if k < K - 1:
                plsc.store_scatter(slab, [mi, tokrel], negv)
                nm, nmi = negv, zeroi
                sbase = sseg * _SEGW
                for j in range(_SEGW):
                    v = plsc.load_gather(slab, [sbase + j, tokrel])
                    gt = v > nm
                    nm = jnp.where(gt, v, nm)
                    nmi = jnp.where(gt, sbase + j, nmi)
                for s in range(_NSEG):
                    hit = sseg == s
                    seg_m[s] = jnp.where(hit, nm, seg_m[s])
                    seg_mi[s] = jnp.where(hit, nmi, seg_mi[s])
        es = [jnp.exp(m - ms[0]) for m in ms]
        ssum = es[0]
        for k in range(1, K):
            ssum = ssum + es[k]
        r = 1.0 / ssum
        for k in range(K):
            wk = es[k] * r
            wv[k, pl.ds(goff, _L)] = wk
            iv[k, pl.ds(goff, _L)] = mis[k]
            plsc.store_scatter(maskv, [mis[k] * tpb + tokrel], wk)
        return c
    lax.fori_loop(0, tpb // _L, group, 0)

    pltpu.sync_copy(wv, w_hbm.at[:, wid])
    pltpu.sync_copy(maskv, mask_hbm.at[wid])
    pltpu.sync_copy(iv, idx_hbm.at[:, wid])


def _sc_router(lg3, tpb):
    mesh = plsc.VectorSubcoreMesh(core_axis_name="c", subcore_axis_name="s")
    return pl.kernel(
        functools.partial(_sc_router_body, tpb),
        out_type=[
            jax.ShapeDtypeStruct((K, _NTILE, tpb), jnp.float32),
            jax.ShapeDtypeStruct((_NTILE, E * tpb), jnp.float32),
            jax.ShapeDtypeStruct((K, _NTILE, tpb), jnp.int32),
        ],
        mesh=mesh,
        compiler_params=pltpu.CompilerParams(needs_layout_passes=False),
        scratch_types=[
            pltpu.VMEM((E, tpb), jnp.float32),
            pltpu.VMEM((E * tpb,), jnp.float32),
            pltpu.VMEM((K, tpb), jnp.float32),
            pltpu.VMEM((K, tpb), jnp.int32),
            pltpu.SemaphoreType.DMA,
        ],
    )(lg3)


@jax.jit
def kernel(x, W, b):
    B, S, D = x.shape
    n = B * S
    nch = n // _CH
    tpb = nch // _NTILE
    xf = x.reshape(n, D)
    b2 = b.reshape(E, 1)
    ws, masks, idxs = [], [], []
    for c in range(_CH):
        lg3 = _tc_logits(xf, W, b2, tpb, c, nch)
        w3, mask3, idx3 = _sc_router(lg3, tpb)
        ws.append(w3)
        masks.append(mask3)
        idxs.append(idx3)
    w = jnp.stack(ws).transpose(0, 2, 3, 1).reshape(B, S, K)
    mask = (jnp.stack(masks).reshape(_CH, _NTILE, E, tpb)
            .transpose(0, 1, 3, 2).reshape(B, S, E))
    idx = jnp.stack(idxs).transpose(0, 2, 3, 1).reshape(B, S, K)
    return (w, mask, idx)


# speedup vs baseline: 2.7116x; 1.1013x over previous
import functools

import jax
import jax.numpy as jnp
from jax import lax
from jax.experimental import pallas as pl
from jax.experimental.pallas import tpu as pltpu
from jax.experimental.pallas import tpu_sc as plsc

E = 64
K = 8
_T = 512
_CH = 4

_NC = 2
_NS = 16
_NTILE = _NC * _NS
_L = 16
_NSEG = 8
_SEGW = E // _NSEG

_NEG = float("-inf")


def _logits_body(x_ref, w_ref, b_ref, out_ref):
    lg = lax.dot_general(
        w_ref[...], x_ref[...], (((1,), (1,)), ((), ())),
        preferred_element_type=jnp.float32)
    lg = lg + b_ref[...]
    tpb = out_ref.shape[2]
    for s in range(out_ref.shape[0]):
        out_ref[s] = lg[:, s * tpb:(s + 1) * tpb]


def _tc_logits(xf, W, b2, tpb, chunk, nch):
    n, D = xf.shape
    grid = nch // _T
    step0 = chunk * grid
    tiles_per_step = _T // tpb
    return pl.pallas_call(
        _logits_body,
        grid=(grid,),
        in_specs=[
            pl.BlockSpec((_T, D), lambda i: (step0 + i, 0)),
            pl.BlockSpec((E, D), lambda i: (0, 0)),
            pl.BlockSpec((E, 1), lambda i: (0, 0)),
        ],
        out_specs=pl.BlockSpec(
            (tiles_per_step, E, tpb), lambda i: (i, 0, 0)),
        out_shape=jax.ShapeDtypeStruct((_NTILE, E, tpb), jnp.float32),
        compiler_params=pltpu.CompilerParams(
            dimension_semantics=("parallel",),
        ),
    )(xf, W, b2)


def _sc_router_body(tpb, lg_hbm, w_hbm, mask_hbm, idx_hbm,
                    slab, maskv, wv, iv, sem):
    wid = lax.axis_index("s") * _NC + lax.axis_index("c")
    cp = pltpu.make_async_copy(lg_hbm.at[wid], slab, sem)
    cp.start()
    zero = jnp.zeros((_L,), jnp.float32)

    def zbody(r, c):
        for j in range(8):
            maskv[pl.ds(r * 8 * _L + j * _L, _L)] = zero
        return c
    lax.fori_loop(0, tpb * E // (8 * _L), zbody, 0)
    cp.wait()

    toki = lax.broadcasted_iota(jnp.int32, (_L,), 0)
    negv = jnp.full((_L,), _NEG, jnp.float32)
    zeroi = jnp.zeros((_L,), jnp.int32)

    def group(g, c):
        goff = g * _L
        tokrel = goff + toki
        seg_m, seg_mi = [], []
        for s in range(_NSEG):
            m, mi = negv, zeroi
            for j in range(_SEGW):
                e = s * _SEGW + j
                v = slab[e, pl.ds(goff, _L)]
                gt = v > m
                m = jnp.where(gt, v, m)
                mi = jnp.where(gt, jnp.full((_L,), e, jnp.int32), mi)
            seg_m.append(m)
            seg_mi.append(mi)

        ms, mis = [], []
        for k in range(K):
            m, mi, sseg = seg_m[0], seg_mi[0], zeroi
            for s in range(1, _NSEG):
                gt = seg_m[s] > m
                m = jnp.where(gt, seg_m[s], m)
                mi = jnp.where(gt, seg_mi[s], mi)
                sseg = jnp.where(gt, jnp.full((_L,), s, jnp.int32), sseg)
            ms.append(m)
            mis.append(mi)
            if k < K - 1:
                plsc.store_scatter(slab, [mi, tokrel], negv)
                nm, nmi = negv, zeroi
                sbase = sseg * _SEGW
                for j in range(_SEGW):
                    v = plsc.load_gather(slab, [sbase + j, tokrel])
                    gt = v > nm
                    nm = jnp.where(gt, v, nm)
                    nmi = jnp.where(gt, sbase + j, nmi)
                for s in range(_NSEG):
                    hit = sseg == s
                    seg_m[s] = jnp.where(hit, nm, seg_m[s])
                    seg_mi[s] = jnp.where(hit, nmi, seg_mi[s])
        es = [jnp.exp(m - ms[0]) for m in ms]
        ssum = es[0]
        for k in range(1, K):
            ssum = ssum + es[k]
        r = 1.0 / ssum
        for k in range(K):
            wk = es[k] * r
            wv[k, pl.ds(goff, _L)] = wk
            iv[k, pl.ds(goff, _L)] = mis[k]
            plsc.store_scatter(maskv, [mis[k] * tpb + tokrel], wk)
        return c
    lax.fori_loop(0, tpb // _L, group, 0)

    pltpu.sync_copy(wv, w_hbm.at[:, wid])
    pltpu.sync_copy(maskv, mask_hbm.at[wid])
    pltpu.sync_copy(iv, idx_hbm.at[:, wid])


def _sc_router(lg3, tpb):
    mesh = plsc.VectorSubcoreMesh(core_axis_name="c", subcore_axis_name="s")
    return pl.kernel(
        functools.partial(_sc_router_body, tpb),
        out_type=[
            jax.ShapeDtypeStruct((K, _NTILE, tpb), jnp.float32),
            jax.ShapeDtypeStruct((_NTILE, E * tpb), jnp.float32),
            jax.ShapeDtypeStruct((K, _NTILE, tpb), jnp.int32),
        ],
        mesh=mesh,
        compiler_params=pltpu.CompilerParams(needs_layout_passes=False),
        scratch_types=[
            pltpu.VMEM((E, tpb), jnp.float32),
            pltpu.VMEM((E * tpb,), jnp.float32),
            pltpu.VMEM((K, tpb), jnp.float32),
            pltpu.VMEM((K, tpb), jnp.int32),
            pltpu.SemaphoreType.DMA,
        ],
    )(lg3)


@jax.jit
def kernel(x, W, b):
    B, S, D = x.shape
    n = B * S
    nch = n // _CH
    tpb = nch // _NTILE
    xf = x.reshape(n, D)
    b2 = b.reshape(E, 1)
    ws, masks, idxs = [], [], []
    for c in range(_CH):
        lg3 = _tc_logits(xf, W, b2, tpb, c, nch)
        w3, mask3, idx3 = _sc_router(lg3, tpb)
        ws.append(w3)
        masks.append(mask3)
        idxs.append(idx3)
    w = jnp.stack(ws).transpose(0, 2, 3, 1).reshape(B, S, K)
    mask = (jnp.stack(masks).reshape(_CH, _NTILE, E, tpb)
            .transpose(0, 1, 3, 2).reshape(B, S, E))
    idx = jnp.stack(idxs).transpose(0, 2, 3, 1).reshape(B, S, K)
    return (w, mask, idx)
